# Initial kernel scaffold; baseline (speedup 1.0000x reference)
import jax
import jax.numpy as jnp
from jax.experimental import pallas as pl


def kernel(x, batch):
    raise NotImplementedError("write your pallas kernel here")


# speedup vs baseline: 3.1432x; 3.1432x over previous
import functools

import jax
import jax.numpy as jnp
from jax import lax
from jax.experimental import pallas as pl
from jax.experimental.pallas import tpu as pltpu
from jax.experimental.pallas import tpu_sc as plsc

_B = 16
_MAX_NODES = 4096
_N = 32768
_D = 128

_NC = 2
_NS = 16
_NW = _NC * _NS
_ROWS_PER_W = (_B * _MAX_NODES) // _NW
_CHUNK = 256
_BITS = (128, 64, 32, 16, 8, 4, 2, 1)


def _to_dense_batch_sc(x, batch):
    mesh = plsc.VectorSubcoreMesh(core_axis_name="c", subcore_axis_name="s")

    @functools.partial(
        pl.kernel,
        mesh=mesh,
        out_type=jax.ShapeDtypeStruct((_B * _MAX_NODES, _D), jnp.float32),
        scratch_types=[
            pltpu.VMEM((_N,), jnp.int32),
            pltpu.VMEM((_CHUNK, _D), jnp.float32),
            pltpu.VMEM((_CHUNK, _D), jnp.float32),
        ],
        compiler_params=pltpu.CompilerParams(use_tc_tiling_on_sc=False,
                                             needs_layout_passes=False),
    )
    def run(x_hbm, batch_hbm, out_hbm, batch_v, stage_v, zero_v):
        wid = lax.axis_index("s") * _NC + lax.axis_index("c")
        b = wid // 2
        node_base = (wid % 2) * _ROWS_PER_W

        pltpu.sync_copy(batch_hbm, batch_v)

        def zinit(i, carry):
            for j in range(_D // 16):
                zero_v[i, pl.ds(j * 16, 16)] = jnp.zeros((16,), jnp.float32)
            return carry

        lax.fori_loop(0, _CHUNK, zinit, 0)

        def lower_bound(v):
            def step(_, lohi):
                lo, hi = lohi
                mid = (lo + hi) // 2
                row = batch_v[pl.ds(mid * 16, 16)]
                pred = row[0] < v
                return (jnp.where(pred, mid + 1, lo),
                        jnp.where(pred, hi, mid))

            nrows = _N // 16
            r, _ = lax.fori_loop(0, 11, step,
                                 (jnp.int32(0), jnp.int32(nrows)))
            rprev = jnp.maximum(r - 1, 0)
            row = batch_v[pl.ds(rprev * 16, 16)]
            cnt_lt = jnp.sum(jnp.where(row < v, 1, 0).astype(jnp.int32))
            return jnp.where(r == 0, 0, rprev * 16 + cnt_lt)

        seg_lo = lower_bound(b)
        seg_hi = lower_bound(b + 1)
        cnt = jnp.minimum(seg_hi - seg_lo, _MAX_NODES)
        valid = jnp.clip(cnt - node_base, 0, _ROWS_PER_W)

        src0 = seg_lo + node_base
        dst0 = wid * _ROWS_PER_W

        nf = valid // _CHUNK

        def copy_body(i, carry):
            pltpu.sync_copy(x_hbm.at[pl.ds(src0 + i * _CHUNK, _CHUNK)],
                            stage_v)
            pltpu.sync_copy(stage_v,
                            out_hbm.at[pl.ds(dst0 + i * _CHUNK, _CHUNK)])
            return carry

        lax.fori_loop(0, nf, copy_body, 0)

        rem = valid - nf * _CHUNK
        roff_src = src0 + nf * _CHUNK
        roff_dst = dst0 + nf * _CHUNK
        for s in _BITS:
            @pl.when((rem & s) != 0)
            def _(s=s, roff_src=roff_src, roff_dst=roff_dst):
                pltpu.sync_copy(x_hbm.at[pl.ds(roff_src, s)],
                                stage_v.at[pl.ds(0, s)])
                pltpu.sync_copy(stage_v.at[pl.ds(0, s)],
                                out_hbm.at[pl.ds(roff_dst, s)])
            roff_src = roff_src + (rem & s)
            roff_dst = roff_dst + (rem & s)

        zcnt = _ROWS_PER_W - valid
        zdst0 = dst0 + valid
        nzf = zcnt // _CHUNK

        def zero_body(i, carry):
            pltpu.sync_copy(zero_v,
                            out_hbm.at[pl.ds(zdst0 + i * _CHUNK, _CHUNK)])
            return carry

        lax.fori_loop(0, nzf, zero_body, 0)

        zrem = zcnt - nzf * _CHUNK
        zoff = zdst0 + nzf * _CHUNK
        for s in _BITS:
            @pl.when((zrem & s) != 0)
            def _(s=s, zoff=zoff):
                pltpu.sync_copy(zero_v.at[pl.ds(0, s)],
                                out_hbm.at[pl.ds(zoff, s)])
            zoff = zoff + (zrem & s)

    return run(x, batch)


def kernel(x, batch):
    out = _to_dense_batch_sc(x, batch.astype(jnp.int32))
    return out.reshape(_B, _MAX_NODES, _D)


# speedup vs baseline: 3.5841x; 1.1403x over previous
import functools

import jax
import jax.numpy as jnp
from jax import lax
from jax.experimental import pallas as pl
from jax.experimental.pallas import tpu as pltpu
from jax.experimental.pallas import tpu_sc as plsc

_B = 16
_MAX_NODES = 4096
_N = 32768
_D = 128

_NC = 2
_NS = 16
_NW = _NC * _NS
_ROWS_PER_W = (_B * _MAX_NODES) // _NW
_CHUNK = 128
_CHUNKZ = 256
_BITS = (128, 64, 32, 16, 8, 4, 2, 1)


def _to_dense_batch_sc(x, batch):
    mesh = plsc.VectorSubcoreMesh(core_axis_name="c", subcore_axis_name="s")

    @functools.partial(
        pl.kernel,
        mesh=mesh,
        out_type=jax.ShapeDtypeStruct((_B * _MAX_NODES, _D), jnp.float32),
        scratch_types=[
            pltpu.VMEM((_N,), jnp.int32),
            pltpu.VMEM((_CHUNK, _D), jnp.float32),
            pltpu.VMEM((_CHUNK, _D), jnp.float32),
            pltpu.VMEM((_CHUNKZ, _D), jnp.float32),
            pltpu.SemaphoreType.DMA,
            pltpu.SemaphoreType.DMA,
            pltpu.SemaphoreType.DMA,
            pltpu.SemaphoreType.DMA,
            pltpu.SemaphoreType.DMA,
            pltpu.SemaphoreType.DMA,
        ],
        compiler_params=pltpu.CompilerParams(use_tc_tiling_on_sc=False,
                                             needs_layout_passes=False),
    )
    def run(x_hbm, batch_hbm, out_hbm, batch_v, stage0, stage1, zero_v,
            bsem, rd0, rd1, wr0, wr1, zsem):
        wid = lax.axis_index("s") * _NC + lax.axis_index("c")
        b = wid // 2
        node_base = (wid % 2) * _ROWS_PER_W

        batch_cp = pltpu.async_copy(batch_hbm, batch_v, bsem)

        def zinit(i, carry):
            for j in range(_D // 16):
                zero_v[i, pl.ds(j * 16, 16)] = jnp.zeros((16,), jnp.float32)
            return carry

        lax.fori_loop(0, _CHUNKZ, zinit, 0)
        batch_cp.wait()

        def lower_bound(v):
            def step(_, lohi):
                lo, hi = lohi
                mid = (lo + hi) // 2
                row = batch_v[pl.ds(mid * 16, 16)]
                pred = row[0] < v
                return (jnp.where(pred, mid + 1, lo),
                        jnp.where(pred, hi, mid))

            nrows = _N // 16
            r, _ = lax.fori_loop(0, 11, step,
                                 (jnp.int32(0), jnp.int32(nrows)))
            rprev = jnp.maximum(r - 1, 0)
            row = batch_v[pl.ds(rprev * 16, 16)]
            cnt_lt = jnp.sum(jnp.where(row < v, 1, 0).astype(jnp.int32))
            return jnp.where(r == 0, 0, rprev * 16 + cnt_lt)

        seg_lo = lower_bound(b)
        seg_hi = lower_bound(b + 1)
        cnt = jnp.minimum(seg_hi - seg_lo, _MAX_NODES)
        valid = jnp.clip(cnt - node_base, 0, _ROWS_PER_W)

        src0 = seg_lo + node_base
        dst0 = wid * _ROWS_PER_W

        zcnt = _ROWS_PER_W - valid
        zdst0 = dst0 + valid
        nzf = zcnt // _CHUNKZ

        def zbody(i, carry):
            pltpu.async_copy(
                zero_v, out_hbm.at[pl.ds(zdst0 + i * _CHUNKZ, _CHUNKZ)], zsem)
            return carry

        lax.fori_loop(0, nzf, zbody, 0)

        nf = valid // _CHUNK
        stages = (stage0, stage1)
        rds = (rd0, rd1)
        wrs = (wr0, wr1)
        npair = (nf + 2) // 2

        def pbody(g, carry):
            for hb in range(2):
                i = 2 * g + hb
                st, rs, ws = stages[hb], rds[hb], wrs[hb]

                @pl.when(jnp.logical_and(i >= 2, i < nf))
                def _():
                    pltpu.make_async_copy(
                        st, out_hbm.at[pl.ds(dst0, _CHUNK)], ws).wait()

                @pl.when(i < nf)
                def _():
                    pltpu.async_copy(
                        x_hbm.at[pl.ds(src0 + i * _CHUNK, _CHUNK)], st, rs)

                po = 1 - hb
                pst, prs, pws = stages[po], rds[po], wrs[po]
                im1 = i - 1

                @pl.when(jnp.logical_and(im1 >= 0, im1 < nf))
                def _():
                    pltpu.make_async_copy(
                        x_hbm.at[pl.ds(src0, _CHUNK)], pst, prs).wait()
                    pltpu.async_copy(
                        pst,
                        out_hbm.at[pl.ds(dst0 + im1 * _CHUNK, _CHUNK)], pws)
            return carry

        lax.fori_loop(0, npair, pbody, 0)

        @pl.when(nf >= 1)
        def _():
            pltpu.make_async_copy(
                stage0, out_hbm.at[pl.ds(dst0, _CHUNK)], wr0).wait()

        @pl.when(nf >= 2)
        def _():
            pltpu.make_async_copy(
                stage1, out_hbm.at[pl.ds(dst0, _CHUNK)], wr1).wait()

        rem = valid - nf * _CHUNK
        roff_src = src0 + nf * _CHUNK
        roff_dst = dst0 + nf * _CHUNK
        for s in _BITS:
            @pl.when((rem & s) != 0)
            def _(s=s, roff_src=roff_src, roff_dst=roff_dst):
                pltpu.sync_copy(x_hbm.at[pl.ds(roff_src, s)],
                                stage0.at[pl.ds(0, s)])
                pltpu.sync_copy(stage0.at[pl.ds(0, s)],
                                out_hbm.at[pl.ds(roff_dst, s)])
            roff_src = roff_src + (rem & s)
            roff_dst = roff_dst + (rem & s)

        zrem = zcnt - nzf * _CHUNKZ
        zoff = zdst0 + nzf * _CHUNKZ
        for s in _BITS:
            @pl.when((zrem & s) != 0)
            def _(s=s, zoff=zoff):
                pltpu.sync_copy(zero_v.at[pl.ds(0, s)],
                                out_hbm.at[pl.ds(zoff, s)])
            zoff = zoff + (zrem & s)

        def zdrain(i, carry):
            pltpu.make_async_copy(
                zero_v, out_hbm.at[pl.ds(zdst0, _CHUNKZ)], zsem).wait()
            return carry

        lax.fori_loop(0, nzf, zdrain, 0)

    return run(x, batch)


def kernel(x, batch):
    out = _to_dense_batch_sc(x, batch.astype(jnp.int32))
    return out.reshape(_B, _MAX_NODES, _D)


# speedup vs baseline: 3.6147x; 1.0086x over previous
import functools

import jax
import jax.numpy as jnp
from jax import lax
from jax.experimental import pallas as pl
from jax.experimental.pallas import tpu as pltpu
from jax.experimental.pallas import tpu_sc as plsc

_B = 16
_MAX_NODES = 4096
_N = 32768
_D = 128

_NC = 2
_NS = 16
_NW = _NC * _NS
_ROWS_PER_W = (_B * _MAX_NODES) // _NW
_CHUNK = 128
_CHUNKZ = 256
_BITS = (128, 64, 32, 16, 8, 4, 2, 1)


def _to_dense_batch_sc(x, batch):
    mesh = plsc.VectorSubcoreMesh(core_axis_name="c", subcore_axis_name="s")

    @functools.partial(
        pl.kernel,
        mesh=mesh,
        out_type=jax.ShapeDtypeStruct((_B * _MAX_NODES, _D), jnp.float32),
        scratch_types=[
            pltpu.VMEM((_N,), jnp.int32),
            pltpu.VMEM((_CHUNK, _D), jnp.float32),
            pltpu.VMEM((_CHUNK, _D), jnp.float32),
            pltpu.VMEM((_CHUNKZ, _D), jnp.float32),
            pltpu.SemaphoreType.DMA,
            pltpu.SemaphoreType.DMA,
            pltpu.SemaphoreType.DMA,
            pltpu.SemaphoreType.DMA,
            pltpu.SemaphoreType.DMA,
            pltpu.SemaphoreType.DMA,
        ],
        compiler_params=pltpu.CompilerParams(use_tc_tiling_on_sc=False,
                                             needs_layout_passes=False),
    )
    def run(x_hbm, batch_hbm, out_hbm, batch_v, stage0, stage1, zero_v,
            bsem, rd0, rd1, wr0, wr1, zsem):
        wid = lax.axis_index("s") * _NC + lax.axis_index("c")
        b = wid // 2
        node_base = ((wid % 2) ^ (b % 2)) * _ROWS_PER_W

        batch_cp = pltpu.async_copy(batch_hbm, batch_v, bsem)

        def zinit(i, carry):
            for j in range(_D // 16):
                zero_v[i, pl.ds(j * 16, 16)] = jnp.zeros((16,), jnp.float32)
            return carry

        lax.fori_loop(0, _CHUNKZ, zinit, 0)
        batch_cp.wait()

        def lower_bound(v):
            def step(_, lohi):
                lo, hi = lohi
                mid = (lo + hi) // 2
                row = batch_v[pl.ds(mid * 16, 16)]
                pred = row[0] < v
                return (jnp.where(pred, mid + 1, lo),
                        jnp.where(pred, hi, mid))

            nrows = _N // 16
            r, _ = lax.fori_loop(0, 11, step,
                                 (jnp.int32(0), jnp.int32(nrows)))
            rprev = jnp.maximum(r - 1, 0)
            row = batch_v[pl.ds(rprev * 16, 16)]
            cnt_lt = jnp.sum(jnp.where(row < v, 1, 0).astype(jnp.int32))
            return jnp.where(r == 0, 0, rprev * 16 + cnt_lt)

        seg_lo = lower_bound(b)
        seg_hi = lower_bound(b + 1)
        cnt = jnp.minimum(seg_hi - seg_lo, _MAX_NODES)
        valid = jnp.clip(cnt - node_base, 0, _ROWS_PER_W)

        src0 = seg_lo + node_base
        dst0 = b * _MAX_NODES + node_base

        zcnt = _ROWS_PER_W - valid
        zdst0 = dst0 + valid
        nzf = zcnt // _CHUNKZ

        def zbody(i, carry):
            pltpu.async_copy(
                zero_v, out_hbm.at[pl.ds(zdst0 + i * _CHUNKZ, _CHUNKZ)], zsem)
            return carry

        lax.fori_loop(0, nzf, zbody, 0)

        nf = valid // _CHUNK
        stages = (stage0, stage1)
        rds = (rd0, rd1)
        wrs = (wr0, wr1)
        npair = (nf + 2) // 2

        def pbody(g, carry):
            for hb in range(2):
                i = 2 * g + hb
                st, rs, ws = stages[hb], rds[hb], wrs[hb]

                @pl.when(jnp.logical_and(i >= 2, i < nf))
                def _():
                    pltpu.make_async_copy(
                        st, out_hbm.at[pl.ds(dst0, _CHUNK)], ws).wait()

                @pl.when(i < nf)
                def _():
                    pltpu.async_copy(
                        x_hbm.at[pl.ds(src0 + i * _CHUNK, _CHUNK)], st, rs)

                po = 1 - hb
                pst, prs, pws = stages[po], rds[po], wrs[po]
                im1 = i - 1

                @pl.when(jnp.logical_and(im1 >= 0, im1 < nf))
                def _():
                    pltpu.make_async_copy(
                        x_hbm.at[pl.ds(src0, _CHUNK)], pst, prs).wait()
                    pltpu.async_copy(
                        pst,
                        out_hbm.at[pl.ds(dst0 + im1 * _CHUNK, _CHUNK)], pws)
            return carry

        lax.fori_loop(0, npair, pbody, 0)

        @pl.when(nf >= 1)
        def _():
            pltpu.make_async_copy(
                stage0, out_hbm.at[pl.ds(dst0, _CHUNK)], wr0).wait()

        @pl.when(nf >= 2)
        def _():
            pltpu.make_async_copy(
                stage1, out_hbm.at[pl.ds(dst0, _CHUNK)], wr1).wait()

        rem = valid - nf * _CHUNK
        roff_src = src0 + nf * _CHUNK
        roff_dst = dst0 + nf * _CHUNK
        for s in _BITS:
            @pl.when((rem & s) != 0)
            def _(s=s, roff_src=roff_src, roff_dst=roff_dst):
                pltpu.sync_copy(x_hbm.at[pl.ds(roff_src, s)],
                                stage0.at[pl.ds(0, s)])
                pltpu.sync_copy(stage0.at[pl.ds(0, s)],
                                out_hbm.at[pl.ds(roff_dst, s)])
            roff_src = roff_src + (rem & s)
            roff_dst = roff_dst + (rem & s)

        zrem = zcnt - nzf * _CHUNKZ
        zoff = zdst0 + nzf * _CHUNKZ
        for s in _BITS:
            @pl.when((zrem & s) != 0)
            def _(s=s, zoff=zoff):
                pltpu.sync_copy(zero_v.at[pl.ds(0, s)],
                                out_hbm.at[pl.ds(zoff, s)])
            zoff = zoff + (zrem & s)

        def zdrain(i, carry):
            pltpu.make_async_copy(
                zero_v, out_hbm.at[pl.ds(zdst0, _CHUNKZ)], zsem).wait()
            return carry

        lax.fori_loop(0, nzf, zdrain, 0)

    return run(x, batch)


def kernel(x, batch):
    out = _to_dense_batch_sc(x, batch.astype(jnp.int32))
    return out.reshape(_B, _MAX_NODES, _D)


# speedup vs baseline: 3.8666x; 1.0697x over previous
import functools

import jax
import jax.numpy as jnp
from jax import lax
from jax.experimental import pallas as pl
from jax.experimental.pallas import tpu as pltpu
from jax.experimental.pallas import tpu_sc as plsc

_B = 16
_MAX_NODES = 4096
_N = 32768
_D = 128

_NC = 2
_NS = 16
_NW = _NC * _NS
_QROWS = _MAX_NODES // 4
_CHUNK = 256
_CHUNKZ = 192
_BITS = (128, 64, 32, 16, 8, 4, 2, 1)


def _to_dense_batch_sc(x, batch):
    mesh = plsc.VectorSubcoreMesh(core_axis_name="c", subcore_axis_name="s")

    @functools.partial(
        pl.kernel,
        mesh=mesh,
        out_type=jax.ShapeDtypeStruct((_B * _MAX_NODES, _D), jnp.float32),
        scratch_types=[
            pltpu.VMEM((_N,), jnp.int32),
            pltpu.VMEM((_CHUNK, _D), jnp.float32),
            pltpu.VMEM((_CHUNK, _D), jnp.float32),
            pltpu.VMEM((_CHUNKZ, _D), jnp.float32),
            pltpu.SemaphoreType.DMA,
            pltpu.SemaphoreType.DMA,
            pltpu.SemaphoreType.DMA,
            pltpu.SemaphoreType.DMA,
            pltpu.SemaphoreType.DMA,
            pltpu.SemaphoreType.DMA,
        ],
        compiler_params=pltpu.CompilerParams(use_tc_tiling_on_sc=False,
                                             needs_layout_passes=False),
    )
    def run(x_hbm, batch_hbm, out_hbm, batch_v, stage0, stage1, zero_v,
            bsem, rd0, rd1, wr0, wr1, zsem):
        wid = lax.axis_index("s") * _NC + lax.axis_index("c")
        b = wid // 2
        qa = wid % 2

        batch_cp = pltpu.async_copy(batch_hbm, batch_v, bsem)

        def zinit(i, carry):
            for j in range(_D // 16):
                zero_v[i, pl.ds(j * 16, 16)] = jnp.zeros((16,), jnp.float32)
            return carry

        lax.fori_loop(0, _CHUNKZ, zinit, 0)
        batch_cp.wait()

        def lower_bound(v):
            def step(_, lohi):
                lo, hi = lohi
                mid = (lo + hi) // 2
                row = batch_v[pl.ds(mid * 16, 16)]
                pred = row[0] < v
                return (jnp.where(pred, mid + 1, lo),
                        jnp.where(pred, hi, mid))

            nrows = _N // 16
            r, _ = lax.fori_loop(0, 11, step,
                                 (jnp.int32(0), jnp.int32(nrows)))
            rprev = jnp.maximum(r - 1, 0)
            row = batch_v[pl.ds(rprev * 16, 16)]
            cnt_lt = jnp.sum(jnp.where(row < v, 1, 0).astype(jnp.int32))
            return jnp.where(r == 0, 0, rprev * 16 + cnt_lt)

        seg_lo = lower_bound(b)
        seg_hi = lower_bound(b + 1)
        cnt = jnp.minimum(seg_hi - seg_lo, _MAX_NODES)

        regions = []
        for q in (qa, 3 - qa):
            node_lo = q * _QROWS
            valid = jnp.clip(cnt - node_lo, 0, _QROWS)
            src0 = seg_lo + node_lo
            dst0 = b * _MAX_NODES + node_lo
            regions.append((valid, src0, dst0))

        zstate = []
        for valid, _src0, dst0 in regions:
            zcnt = _QROWS - valid
            zdst0 = dst0 + valid
            nzf = zcnt // _CHUNKZ

            def zbody(i, carry, zdst0=zdst0):
                pltpu.async_copy(
                    zero_v,
                    out_hbm.at[pl.ds(zdst0 + i * _CHUNKZ, _CHUNKZ)], zsem)
                return carry

            lax.fori_loop(0, nzf, zbody, 0)
            zstate.append((nzf, zcnt - nzf * _CHUNKZ, zdst0 + nzf * _CHUNKZ))

        stages = (stage0, stage1)
        rds = (rd0, rd1)
        wrs = (wr0, wr1)

        for valid, src0, dst0 in regions:
            nf = valid // _CHUNK
            npair = (nf + 2) // 2

            def pbody(g, carry, src0=src0, dst0=dst0, nf=nf):
                for hb in range(2):
                    i = 2 * g + hb
                    st, rs, ws = stages[hb], rds[hb], wrs[hb]

                    @pl.when(jnp.logical_and(i >= 2, i < nf))
                    def _():
                        pltpu.make_async_copy(
                            st, out_hbm.at[pl.ds(dst0, _CHUNK)], ws).wait()

                    @pl.when(i < nf)
                    def _():
                        pltpu.async_copy(
                            x_hbm.at[pl.ds(src0 + i * _CHUNK, _CHUNK)],
                            st, rs)

                    po = 1 - hb
                    pst, prs, pws = stages[po], rds[po], wrs[po]
                    im1 = i - 1

                    @pl.when(jnp.logical_and(im1 >= 0, im1 < nf))
                    def _():
                        pltpu.make_async_copy(
                            x_hbm.at[pl.ds(src0, _CHUNK)], pst, prs).wait()
                        pltpu.async_copy(
                            pst,
                            out_hbm.at[pl.ds(dst0 + im1 * _CHUNK, _CHUNK)],
                            pws)
                return carry

            lax.fori_loop(0, npair, pbody, 0)

            @pl.when(nf >= 1)
            def _(dst0=dst0):
                pltpu.make_async_copy(
                    stage0, out_hbm.at[pl.ds(dst0, _CHUNK)], wr0).wait()

            @pl.when(nf >= 2)
            def _(dst0=dst0):
                pltpu.make_async_copy(
                    stage1, out_hbm.at[pl.ds(dst0, _CHUNK)], wr1).wait()

            rem = valid - nf * _CHUNK
            roff_src = src0 + nf * _CHUNK
            roff_dst = dst0 + nf * _CHUNK
            for s in _BITS:
                @pl.when((rem & s) != 0)
                def _(s=s, roff_src=roff_src, roff_dst=roff_dst):
                    pltpu.sync_copy(x_hbm.at[pl.ds(roff_src, s)],
                                    stage0.at[pl.ds(0, s)])
                    pltpu.sync_copy(stage0.at[pl.ds(0, s)],
                                    out_hbm.at[pl.ds(roff_dst, s)])
                roff_src = roff_src + (rem & s)
                roff_dst = roff_dst + (rem & s)

        for _nzf, zrem, zoff in zstate:
            for s in _BITS:
                @pl.when((zrem & s) != 0)
                def _(s=s, zoff=zoff):
                    pltpu.sync_copy(zero_v.at[pl.ds(0, s)],
                                    out_hbm.at[pl.ds(zoff, s)])
                zoff = zoff + (zrem & s)

        nz_total = zstate[0][0] + zstate[1][0]
        zdrain_dst = regions[0][2]

        def zdrain(i, carry):
            pltpu.make_async_copy(
                zero_v, out_hbm.at[pl.ds(zdrain_dst, _CHUNKZ)], zsem).wait()
            return carry

        lax.fori_loop(0, nz_total, zdrain, 0)

    return run(x, batch)


def kernel(x, batch):
    out = _to_dense_batch_sc(x, batch.astype(jnp.int32))
    return out.reshape(_B, _MAX_NODES, _D)
